# Initial kernel scaffold; baseline (speedup 1.0000x reference)
import jax
import jax.numpy as jnp
from jax.experimental import pallas as pl


def kernel(x, batch, learnable_param, last_updated_param):
    raise NotImplementedError("write your pallas kernel here")


# speedup vs baseline: 19.5518x; 19.5518x over previous
import functools

import jax
import jax.numpy as jnp
from jax.experimental import pallas as pl


def _body(x_ref, lup_ref, lp_ref, bc_ref, o_ref):
    xb = x_ref[...]
    cand = xb[:, 0:1] == bc_ref[...]
    hit = jnp.any(cand)

    @pl.when(jnp.logical_not(hit))
    def _copy():
        o_ref[...] = xb

    @pl.when(hit)
    def _full():
        o = xb
        for p in range(lup_ref.shape[0]):
            m = jnp.all(o == lup_ref[p : p + 1, :], axis=1, keepdims=True)
            o = jnp.where(m, lp_ref[p : p + 1, :], o)
        o_ref[...] = o


@jax.jit
def kernel(x, batch, learnable_param, last_updated_param):
    del batch
    n, d = x.shape
    p = last_updated_param.shape[0]
    blk = 2048
    grid = (n + blk - 1) // blk
    bcol = last_updated_param[:, 0:1].T

    return pl.pallas_call(
        _body,
        grid=(grid,),
        in_specs=[
            pl.BlockSpec((blk, d), lambda i: (i, 0)),
            pl.BlockSpec((p, d), lambda i: (0, 0)),
            pl.BlockSpec((p, d), lambda i: (0, 0)),
            pl.BlockSpec((1, p), lambda i: (0, 0)),
        ],
        out_specs=pl.BlockSpec((blk, d), lambda i: (i, 0)),
        out_shape=jax.ShapeDtypeStruct((n, d), x.dtype),
    )(x, last_updated_param, learnable_param, bcol)
